# baseline (device time: 199806 ns/iter reference)
import jax
import jax.numpy as jnp
from jax import lax
from jax.experimental import pallas as pl
from jax.experimental.pallas import tpu as pltpu

N_DEV = 8


def kernel(x, w_mat, scale_x, scale_w):
    m_per, k = x.shape
    _, n_per = w_mat.shape

    x8 = x.astype(jnp.float8_e4m3fn)
    w_bf = w_mat.astype(jnp.bfloat16)
    s = (scale_x.astype(jnp.float32) * scale_w.astype(jnp.float32)).reshape(1, 1)

    def body(x_ref, w_ref, s_ref, out_ref, xs_ref, send_sems, recv_sems):
        my = lax.axis_index("i")
        left = lax.rem(my + N_DEV - 1, N_DEV)
        right = lax.rem(my + 1, N_DEV)

        barrier_sem = pltpu.get_barrier_semaphore()
        for nbr in (left, right):
            pl.semaphore_signal(
                barrier_sem, inc=1,
                device_id=(nbr,), device_id_type=pl.DeviceIdType.MESH,
            )
        pl.semaphore_wait(barrier_sem, 2)

        sc = s_ref[0, 0]
        w = w_ref[:, :]

        def compute(c):
            a = xs_ref[c].astype(jnp.bfloat16)
            y = jnp.dot(a, w, preferred_element_type=jnp.float32) * sc
            out_ref[pl.ds(c * m_per, m_per), :] = y * jax.nn.sigmoid(y)

        xs_ref[my] = x_ref[:, :]
        compute(my)

        for h in range(N_DEV - 1):
            c_send = lax.rem(my + N_DEV - h, N_DEV)
            c_recv = lax.rem(my + N_DEV - h - 1, N_DEV)
            rdma = pltpu.make_async_remote_copy(
                src_ref=xs_ref.at[c_send],
                dst_ref=xs_ref.at[c_send],
                send_sem=send_sems.at[h],
                recv_sem=recv_sems.at[h],
                device_id=(right,),
                device_id_type=pl.DeviceIdType.MESH,
            )
            rdma.start()
            rdma.wait()
            compute(c_recv)

    return pl.pallas_call(
        body,
        out_shape=jax.ShapeDtypeStruct((N_DEV * m_per, n_per), jnp.float32),
        in_specs=[
            pl.BlockSpec(memory_space=pltpu.VMEM),
            pl.BlockSpec(memory_space=pltpu.VMEM),
            pl.BlockSpec(memory_space=pltpu.SMEM),
        ],
        out_specs=pl.BlockSpec(memory_space=pltpu.VMEM),
        scratch_shapes=[
            pltpu.VMEM((N_DEV, m_per, k), jnp.float8_e4m3fn),
            pltpu.SemaphoreType.DMA((N_DEV - 1,)),
            pltpu.SemaphoreType.DMA((N_DEV - 1,)),
        ],
        compiler_params=pltpu.CompilerParams(collective_id=0),
    )(x8, w_bf, s)


# device time: 93959 ns/iter; 2.1265x vs baseline; 2.1265x over previous
import jax
import jax.numpy as jnp
from jax import lax
from jax.experimental import pallas as pl
from jax.experimental.pallas import tpu as pltpu

N_DEV = 8
N_PIECE = 2


def kernel(x, w_mat, scale_x, scale_w):
    m_per, k = x.shape
    _, n_per = w_mat.shape
    half = m_per // 2
    piece = half // N_PIECE

    x8 = x.astype(jnp.float8_e4m3fn)
    w_bf = w_mat.astype(jnp.bfloat16)
    s = (scale_x.astype(jnp.float32) * scale_w.astype(jnp.float32)).reshape(1, 1)

    def body(x_ref, w_ref, s_ref, out_ref, xs_ref, send_sems, recv_sems):
        my = lax.axis_index("i")
        left = lax.rem(my + N_DEV - 1, N_DEV)
        right = lax.rem(my + 1, N_DEV)

        barrier_sem = pltpu.get_barrier_semaphore()
        for nbr in (left, right):
            pl.semaphore_signal(
                barrier_sem, inc=1,
                device_id=(nbr,), device_id_type=pl.DeviceIdType.MESH,
            )
        pl.semaphore_wait(barrier_sem, 2)

        sc = s_ref[0, 0]
        w = w_ref[:, :]

        def compute_half(c, d):
            a = xs_ref[c, pl.ds(d * half, half), :].astype(jnp.bfloat16)
            y = jnp.dot(a, w, preferred_element_type=jnp.float32) * sc
            out_ref[pl.ds(c * m_per + d * half, half), :] = y * jax.nn.sigmoid(y)

        xs_ref[my] = x_ref[:, :]

        dst = (right, left)

        def start_send(d, h, p, c):
            rdma = pltpu.make_async_remote_copy(
                src_ref=xs_ref.at[c, pl.ds(d * half + p * piece, piece), :],
                dst_ref=xs_ref.at[c, pl.ds(d * half + p * piece, piece), :],
                send_sem=send_sems.at[d, h, p],
                recv_sem=recv_sems.at[d, h, p],
                device_id=(dst[d],),
                device_id_type=pl.DeviceIdType.MESH,
            )
            rdma.start()
            return rdma

        sends = {}
        for d in range(2):
            for p in range(N_PIECE):
                sends[(d, 0, p)] = start_send(d, 0, p, my)
        compute_half(my, 0)
        compute_half(my, 1)

        for h in range(N_DEV - 1):
            rc = (lax.rem(my + N_DEV - h - 1, N_DEV),
                  lax.rem(my + h + 1, N_DEV))
            for p in range(N_PIECE):
                for d in range(2):
                    sends[(d, h, p)].wait_recv()
                    if h < N_DEV - 2:
                        sends[(d, h + 1, p)] = start_send(d, h + 1, p, rc[d])
            compute_half(rc[0], 0)
            compute_half(rc[1], 1)

        for key, rdma in sends.items():
            rdma.wait_send()

    return pl.pallas_call(
        body,
        out_shape=jax.ShapeDtypeStruct((N_DEV * m_per, n_per), jnp.float32),
        in_specs=[
            pl.BlockSpec(memory_space=pltpu.VMEM),
            pl.BlockSpec(memory_space=pltpu.VMEM),
            pl.BlockSpec(memory_space=pltpu.SMEM),
        ],
        out_specs=pl.BlockSpec(memory_space=pltpu.VMEM),
        scratch_shapes=[
            pltpu.VMEM((N_DEV, m_per, k), jnp.float8_e4m3fn),
            pltpu.SemaphoreType.DMA((2, N_DEV - 1, N_PIECE)),
            pltpu.SemaphoreType.DMA((2, N_DEV - 1, N_PIECE)),
        ],
        compiler_params=pltpu.CompilerParams(collective_id=0),
    )(x8, w_bf, s)
